# Initial kernel scaffold; baseline (speedup 1.0000x reference)
import jax
import jax.numpy as jnp
from jax.experimental import pallas as pl


def kernel(inputs, emb0, emb1, W0, b0, W1, b1, Wa, ba):
    raise NotImplementedError("write your pallas kernel here")


# speedup vs baseline: 2.8418x; 2.8418x over previous
import functools

import jax
import jax.numpy as jnp
from jax import lax
from jax.experimental import pallas as pl
from jax.experimental.pallas import tpu as pltpu
from jax.experimental.pallas import tpu_sc as plsc

B = 1024
T = 50
N = B * T
D0 = 300
D1 = 100
OUT = 300
D0P = 384
D1P = 128

NC, NS = 2, 16
NW = NC * NS
TOK_PER_W = N // NW
CHUNK = 80
NCHUNK = TOK_PER_W // CHUNK

VB = 2000


def _pad_body(emb0_ref, emb1_ref, out0_ref, out1_ref):
    out0_ref[:, :D0] = emb0_ref[...]
    out0_ref[:, D0:] = jnp.zeros((VB, D0P - D0), jnp.float32)
    out1_ref[:, :D1] = emb1_ref[...]
    out1_ref[:, D1:] = jnp.zeros((VB, D1P - D1), jnp.float32)


def _pad_tables(emb0, emb1):
    V = emb0.shape[0]
    return pl.pallas_call(
        _pad_body,
        grid=(V // VB,),
        in_specs=[
            pl.BlockSpec((VB, D0), lambda i: (i, 0)),
            pl.BlockSpec((VB, D1), lambda i: (i, 0)),
        ],
        out_specs=[
            pl.BlockSpec((VB, D0P), lambda i: (i, 0)),
            pl.BlockSpec((VB, D1P), lambda i: (i, 0)),
        ],
        out_shape=[
            jax.ShapeDtypeStruct((V, D0P), jnp.float32),
            jax.ShapeDtypeStruct((V, D1P), jnp.float32),
        ],
    )(emb0, emb1)


def _sc_gather_body(idx_hbm, emb0_hbm, emb1_hbm, e0_out, e1_out,
                    idx_v, buf0, buf1, s0a, s0b, s1a, s1b):
    wid = lax.axis_index("s") * NC + lax.axis_index("c")
    pltpu.sync_copy(idx_hbm.at[wid], idx_v)
    base = wid * TOK_PER_W
    sems0 = (s0a, s0b)
    sems1 = (s1a, s1b)
    cp0 = [pltpu.async_copy(emb0_hbm.at[idx_v.at[0]], buf0.at[0], s0a), None]
    cp1 = [pltpu.async_copy(emb1_hbm.at[idx_v.at[0]], buf1.at[0], s1a), None]
    for j in range(NCHUNK):
        s = j & 1
        if j + 1 < NCHUNK:
            cp0[1 - s] = pltpu.async_copy(
                emb0_hbm.at[idx_v.at[j + 1]], buf0.at[1 - s], sems0[1 - s])
            cp1[1 - s] = pltpu.async_copy(
                emb1_hbm.at[idx_v.at[j + 1]], buf1.at[1 - s], sems1[1 - s])
        cp0[s].wait()
        cp1[s].wait()
        row = base + j * CHUNK
        pltpu.sync_copy(buf0.at[s], e0_out.at[pl.ds(row, CHUNK)])
        pltpu.sync_copy(buf1.at[s], e1_out.at[pl.ds(row, CHUNK)])


def _sc_gather(idx, emb0p, emb1p):
    mesh = plsc.VectorSubcoreMesh(core_axis_name="c", subcore_axis_name="s")
    fn = pl.kernel(
        _sc_gather_body,
        mesh=mesh,
        out_type=[
            jax.ShapeDtypeStruct((N, D0P), jnp.float32),
            jax.ShapeDtypeStruct((N, D1P), jnp.float32),
        ],
        scratch_types=[
            pltpu.VMEM((NCHUNK, CHUNK), jnp.int32),
            pltpu.VMEM((2, CHUNK, D0P), jnp.float32),
            pltpu.VMEM((2, CHUNK, D1P), jnp.float32),
            pltpu.SemaphoreType.DMA,
            pltpu.SemaphoreType.DMA,
            pltpu.SemaphoreType.DMA,
            pltpu.SemaphoreType.DMA,
        ],
    )
    return fn(idx, emb0p, emb1p)


TB = 1024


def _tc_body(e0_ref, e1_ref, W0_ref, W1_ref, b0_ref, b1_ref, wa_ref, out_ref):
    e0 = e0_ref[...]
    e1 = e1_ref[...]
    p0 = jnp.dot(e0, W0_ref[...], preferred_element_type=jnp.float32) + b0_ref[...]
    p1 = jnp.dot(e1, W1_ref[...], preferred_element_type=jnp.float32) + b1_ref[...]
    wa = wa_ref[...]
    s0 = jnp.sum(p0 * wa, axis=1, keepdims=True)
    s1 = jnp.sum(p1 * wa, axis=1, keepdims=True)
    m = jnp.maximum(s0, s1)
    w0 = jnp.exp(s0 - m)
    w1 = jnp.exp(s1 - m)
    inv = 1.0 / (w0 + w1)
    out_ref[...] = (w0 * p0 + w1 * p1) * inv


def _tc_combine(e0, e1, W0p, b0, W1p, b1, Wa):
    grid = (N // TB,)
    return pl.pallas_call(
        _tc_body,
        grid=grid,
        in_specs=[
            pl.BlockSpec((TB, D0P), lambda i: (i, 0)),
            pl.BlockSpec((TB, D1P), lambda i: (i, 0)),
            pl.BlockSpec((D0P, OUT), lambda i: (0, 0)),
            pl.BlockSpec((D1P, OUT), lambda i: (0, 0)),
            pl.BlockSpec((1, OUT), lambda i: (0, 0)),
            pl.BlockSpec((1, OUT), lambda i: (0, 0)),
            pl.BlockSpec((1, OUT), lambda i: (0, 0)),
        ],
        out_specs=pl.BlockSpec((TB, OUT), lambda i: (i, 0)),
        out_shape=jax.ShapeDtypeStruct((N, OUT), jnp.float32),
    )(e0, e1, W0p, W1p, b0, b1, Wa)


def kernel(inputs, emb0, emb1, W0, b0, W1, b1, Wa, ba):
    idx = inputs.reshape(NW, NCHUNK, CHUNK).astype(jnp.int32)
    emb0p, emb1p = _pad_tables(emb0, emb1)
    e0, e1 = _sc_gather(idx, emb0p, emb1p)
    W0p = jnp.pad(W0, ((0, D0P - D0), (0, 0)))
    W1p = jnp.pad(W1, ((0, D1P - D1), (0, 0)))
    out = _tc_combine(
        e0, e1, W0p, b0.reshape(1, OUT), W1p, b1.reshape(1, OUT),
        Wa.reshape(1, OUT),
    )
    return out.reshape(B, T, OUT)


# speedup vs baseline: 3.2561x; 1.1458x over previous
import functools

import jax
import jax.numpy as jnp
from jax import lax
from jax.experimental import pallas as pl
from jax.experimental.pallas import tpu as pltpu
from jax.experimental.pallas import tpu_sc as plsc

B = 1024
T = 50
N = B * T
V = 100000
D0 = 300
D1 = 100
OUT = 300
OUTP = 384

NC, NS = 2, 16
NW = NC * NS
TOK_PER_W = N // NW
CHUNK = 64
NCHUNK = TOK_PER_W // CHUNK

VB = 1024


def _proj_body(e0t_ref, e1t_ref, W0p_ref, W1p_ref, b0p_ref, b1p_ref,
               out0_ref, out1_ref):
    cdims = (((0,), (0,)), ((), ()))
    out0_ref[...] = lax.dot_general(
        e0t_ref[...], W0p_ref[...], cdims,
        preferred_element_type=jnp.float32) + b0p_ref[...]
    out1_ref[...] = lax.dot_general(
        e1t_ref[...], W1p_ref[...], cdims,
        preferred_element_type=jnp.float32) + b1p_ref[...]


def _tc_project(emb0t, emb1t, W0p, W1p, b0p, b1p):
    grid = (pl.cdiv(V, VB),)
    return pl.pallas_call(
        _proj_body,
        grid=grid,
        in_specs=[
            pl.BlockSpec((D0, VB), lambda i: (0, i)),
            pl.BlockSpec((D1, VB), lambda i: (0, i)),
            pl.BlockSpec((D0, OUTP), lambda i: (0, 0)),
            pl.BlockSpec((D1, OUTP), lambda i: (0, 0)),
            pl.BlockSpec((1, OUTP), lambda i: (0, 0)),
            pl.BlockSpec((1, OUTP), lambda i: (0, 0)),
        ],
        out_specs=[
            pl.BlockSpec((VB, OUTP), lambda i: (i, 0)),
            pl.BlockSpec((VB, OUTP), lambda i: (i, 0)),
        ],
        out_shape=[
            jax.ShapeDtypeStruct((V, OUTP), jnp.float32),
            jax.ShapeDtypeStruct((V, OUTP), jnp.float32),
        ],
    )(emb0t, emb1t, W0p, W1p, b0p, b1p)


def _sc_gather_body(idx_hbm, p0_hbm, p1_hbm, g0_out, g1_out,
                    idx_v, buf0, buf1, s0a, s0b, s1a, s1b):
    wid = lax.axis_index("s") * NC + lax.axis_index("c")
    pltpu.sync_copy(idx_hbm.at[wid], idx_v)
    base = wid * TOK_PER_W
    sems0 = (s0a, s0b)
    sems1 = (s1a, s1b)
    cp0 = [pltpu.async_copy(p0_hbm.at[idx_v.at[0]], buf0.at[0], s0a), None]
    cp1 = [pltpu.async_copy(p1_hbm.at[idx_v.at[0]], buf1.at[0], s1a), None]
    for j in range(NCHUNK):
        s = j & 1
        if j + 1 < NCHUNK:
            cp0[1 - s] = pltpu.async_copy(
                p0_hbm.at[idx_v.at[j + 1]], buf0.at[1 - s], sems0[1 - s])
            cp1[1 - s] = pltpu.async_copy(
                p1_hbm.at[idx_v.at[j + 1]], buf1.at[1 - s], sems1[1 - s])
        cp0[s].wait()
        cp1[s].wait()
        row = base + j * CHUNK
        pltpu.sync_copy(buf0.at[s], g0_out.at[pl.ds(row, CHUNK)])
        pltpu.sync_copy(buf1.at[s], g1_out.at[pl.ds(row, CHUNK)])


def _sc_gather(idx, p0, p1):
    mesh = plsc.VectorSubcoreMesh(core_axis_name="c", subcore_axis_name="s")
    fn = pl.kernel(
        _sc_gather_body,
        mesh=mesh,
        out_type=[
            jax.ShapeDtypeStruct((N, OUTP), jnp.float32),
            jax.ShapeDtypeStruct((N, OUTP), jnp.float32),
        ],
        scratch_types=[
            pltpu.VMEM((NCHUNK, CHUNK), jnp.int32),
            pltpu.VMEM((2, CHUNK, OUTP), jnp.float32),
            pltpu.VMEM((2, CHUNK, OUTP), jnp.float32),
            pltpu.SemaphoreType.DMA,
            pltpu.SemaphoreType.DMA,
            pltpu.SemaphoreType.DMA,
            pltpu.SemaphoreType.DMA,
        ],
    )
    return fn(idx, p0, p1)


TB = 1024


def _combine_body(g0_ref, g1_ref, wa_ref, out_ref):
    g0 = g0_ref[...]
    g1 = g1_ref[...]
    wa = wa_ref[...]
    s0 = jnp.sum(g0 * wa, axis=1, keepdims=True)
    s1 = jnp.sum(g1 * wa, axis=1, keepdims=True)
    m = jnp.maximum(s0, s1)
    w0 = jnp.exp(s0 - m)
    w1 = jnp.exp(s1 - m)
    inv = 1.0 / (w0 + w1)
    out_ref[...] = (w0 * g0[:, :OUT] + w1 * g1[:, :OUT]) * inv


def _tc_combine(g0, g1, wap):
    grid = (N // TB,)
    return pl.pallas_call(
        _combine_body,
        grid=grid,
        in_specs=[
            pl.BlockSpec((TB, OUTP), lambda i: (i, 0)),
            pl.BlockSpec((TB, OUTP), lambda i: (i, 0)),
            pl.BlockSpec((1, OUTP), lambda i: (0, 0)),
        ],
        out_specs=pl.BlockSpec((TB, OUT), lambda i: (i, 0)),
        out_shape=jax.ShapeDtypeStruct((N, OUT), jnp.float32),
    )(g0, g1, wap)


def kernel(inputs, emb0, emb1, W0, b0, W1, b1, Wa, ba):
    idx = inputs.reshape(NW, NCHUNK, CHUNK).astype(jnp.int32)
    emb0t = emb0.T
    emb1t = emb1.T
    W0p = jnp.pad(W0, ((0, 0), (0, OUTP - OUT)))
    W1p = jnp.pad(W1, ((0, 0), (0, OUTP - OUT)))
    b0p = jnp.pad(b0, (0, OUTP - OUT)).reshape(1, OUTP)
    b1p = jnp.pad(b1, (0, OUTP - OUT)).reshape(1, OUTP)
    wap = jnp.pad(Wa.reshape(1, OUT), ((0, 0), (0, OUTP - OUT)))
    p0, p1 = _tc_project(emb0t, emb1t, W0p, W1p, b0p, b1p)
    g0, g1 = _sc_gather(idx, p0, p1)
    out = _tc_combine(g0, g1, wap)
    return out.reshape(B, T, OUT)


# speedup vs baseline: 4.5201x; 1.3882x over previous
import functools

import jax
import jax.numpy as jnp
from jax import lax
from jax.experimental import pallas as pl
from jax.experimental.pallas import tpu as pltpu
from jax.experimental.pallas import tpu_sc as plsc

B = 1024
T = 50
N = B * T
V = 100000
D0 = 300
D1 = 100
OUT = 300
OUTP = 384

NC, NS = 2, 16
NW = NC * NS
TOK_PER_W = N // NW
CHUNK = 80
NCHUNK = TOK_PER_W // CHUNK

VB = 1024


def _table_body(e0t_ref, e1t_ref, W0p_ref, W1p_ref, b0p_ref, b1p_ref,
                wa_ref, out_ref):
    cdims = (((0,), (0,)), ((), ()))
    p0 = lax.dot_general(e0t_ref[...], W0p_ref[...], cdims,
                         preferred_element_type=jnp.float32) + b0p_ref[...]
    p1 = lax.dot_general(e1t_ref[...], W1p_ref[...], cdims,
                         preferred_element_type=jnp.float32) + b1p_ref[...]
    wa = wa_ref[...]
    s0 = jnp.sum(p0 * wa, axis=1, keepdims=True)
    s1 = jnp.sum(p1 * wa, axis=1, keepdims=True)
    m = jnp.maximum(s0, s1)
    w0 = jnp.exp(s0 - m)
    w1 = jnp.exp(s1 - m)
    inv = 1.0 / (w0 + w1)
    out_ref[...] = (w0 * p0 + w1 * p1) * inv


def _tc_combined_table(emb0t, emb1t, W0p, W1p, b0p, b1p, wap):
    grid = (pl.cdiv(V, VB),)
    return pl.pallas_call(
        _table_body,
        grid=grid,
        in_specs=[
            pl.BlockSpec((D0, VB), lambda i: (0, i)),
            pl.BlockSpec((D1, VB), lambda i: (0, i)),
            pl.BlockSpec((D0, OUTP), lambda i: (0, 0)),
            pl.BlockSpec((D1, OUTP), lambda i: (0, 0)),
            pl.BlockSpec((1, OUTP), lambda i: (0, 0)),
            pl.BlockSpec((1, OUTP), lambda i: (0, 0)),
            pl.BlockSpec((1, OUTP), lambda i: (0, 0)),
        ],
        out_specs=pl.BlockSpec((VB, OUTP), lambda i: (i, 0)),
        out_shape=jax.ShapeDtypeStruct((V, OUTP), jnp.float32),
    )(emb0t, emb1t, W0p, W1p, b0p, b1p, wap)


def _sc_gather_body(idx_hbm, c_hbm, g_out, idx_v, buf, s0, s1):
    wid = lax.axis_index("s") * NC + lax.axis_index("c")
    pltpu.sync_copy(idx_hbm.at[wid], idx_v)
    base = wid * TOK_PER_W
    sems = (s0, s1)
    cps = [pltpu.async_copy(c_hbm.at[idx_v.at[0]], buf.at[0], s0), None]
    for j in range(NCHUNK):
        s = j & 1
        if j + 1 < NCHUNK:
            cps[1 - s] = pltpu.async_copy(
                c_hbm.at[idx_v.at[j + 1]], buf.at[1 - s], sems[1 - s])
        cps[s].wait()
        pltpu.sync_copy(buf.at[s], g_out.at[pl.ds(base + j * CHUNK, CHUNK)])


def _sc_gather(idx, c):
    mesh = plsc.VectorSubcoreMesh(core_axis_name="c", subcore_axis_name="s")
    fn = pl.kernel(
        _sc_gather_body,
        mesh=mesh,
        out_type=jax.ShapeDtypeStruct((N, OUTP), jnp.float32),
        scratch_types=[
            pltpu.VMEM((NCHUNK, CHUNK), jnp.int32),
            pltpu.VMEM((2, CHUNK, OUTP), jnp.float32),
            pltpu.SemaphoreType.DMA,
            pltpu.SemaphoreType.DMA,
        ],
    )
    return fn(idx, c)


def kernel(inputs, emb0, emb1, W0, b0, W1, b1, Wa, ba):
    idx = inputs.reshape(NW, NCHUNK, CHUNK).astype(jnp.int32)
    emb0t = emb0.T
    emb1t = emb1.T
    W0p = jnp.pad(W0, ((0, 0), (0, OUTP - OUT)))
    W1p = jnp.pad(W1, ((0, 0), (0, OUTP - OUT)))
    b0p = jnp.pad(b0, (0, OUTP - OUT)).reshape(1, OUTP)
    b1p = jnp.pad(b1, (0, OUTP - OUT)).reshape(1, OUTP)
    wap = jnp.pad(Wa.reshape(1, OUT), ((0, 0), (0, OUTP - OUT)))
    c = _tc_combined_table(emb0t, emb1t, W0p, W1p, b0p, b1p, wap)
    g = _sc_gather(idx, c)
    return g[:, :OUT].reshape(B, T, OUT)


# speedup vs baseline: 6.0745x; 1.3439x over previous
import functools

import jax
import jax.numpy as jnp
from jax import lax
from jax.experimental import pallas as pl
from jax.experimental.pallas import tpu as pltpu
from jax.experimental.pallas import tpu_sc as plsc

B = 1024
T = 50
N = B * T
V = 100000
D0 = 300
D1 = 100
OUT = 300
OUTP = 384

NC, NS = 2, 16
NW = NC * NS
CHUNK = T
NCHUNK = B // NW

VB = 2048


def _table_body(e0t_ref, e1t_ref, W0p_ref, W1p_ref, b0p_ref, b1p_ref,
                wa_ref, out_ref):
    cdims = (((0,), (0,)), ((), ()))
    lastdims = (((1,), (1,)), ((), ()))
    p0 = lax.dot_general(e0t_ref[...], W0p_ref[...], cdims,
                         preferred_element_type=jnp.float32) + b0p_ref[...]
    p1 = lax.dot_general(e1t_ref[...], W1p_ref[...], cdims,
                         preferred_element_type=jnp.float32) + b1p_ref[...]
    wa = wa_ref[...]
    s0 = jnp.sum(p0 * wa, axis=1, keepdims=True)
    s1 = jnp.sum(p1 * wa, axis=1, keepdims=True)
    m = jnp.maximum(s0, s1)
    w0 = jnp.exp(s0 - m)
    w1 = jnp.exp(s1 - m)
    inv = 1.0 / (w0 + w1)
    out_ref[...] = (w0 * p0 + w1 * p1) * inv


def _tc_combined_table(emb0t, emb1t, W0p, W1p, b0p, b1p, wap):
    grid = (pl.cdiv(V, VB),)
    return pl.pallas_call(
        _table_body,
        grid=grid,
        in_specs=[
            pl.BlockSpec((D0, VB), lambda i: (0, i)),
            pl.BlockSpec((D1, VB), lambda i: (0, i)),
            pl.BlockSpec((D0, OUTP), lambda i: (0, 0)),
            pl.BlockSpec((D1, OUTP), lambda i: (0, 0)),
            pl.BlockSpec((1, OUTP), lambda i: (0, 0)),
            pl.BlockSpec((1, OUTP), lambda i: (0, 0)),
            pl.BlockSpec((1, OUTP), lambda i: (0, 0)),
        ],
        out_specs=pl.BlockSpec((VB, OUTP), lambda i: (i, 0)),
        out_shape=jax.ShapeDtypeStruct((V, OUTP), jnp.float32),
    )(emb0t, emb1t, W0p, W1p, b0p, b1p, wap)


def _sc_gather_body(idx_hbm, c_hbm, g_out, idx_v, buf, s0, s1):
    wid = lax.axis_index("s") * NC + lax.axis_index("c")
    pltpu.sync_copy(idx_hbm.at[wid], idx_v)
    base = wid * NCHUNK
    sems = (s0, s1)
    cps = [pltpu.async_copy(c_hbm.at[idx_v.at[0]], buf.at[0], s0), None]
    for j in range(NCHUNK):
        s = j & 1
        if j + 1 < NCHUNK:
            cps[1 - s] = pltpu.async_copy(
                c_hbm.at[idx_v.at[j + 1]], buf.at[1 - s], sems[1 - s])
        cps[s].wait()
        pltpu.sync_copy(buf.at[s], g_out.at[base + j])


def _sc_gather(idx, c):
    mesh = plsc.VectorSubcoreMesh(core_axis_name="c", subcore_axis_name="s")
    fn = pl.kernel(
        _sc_gather_body,
        mesh=mesh,
        out_type=jax.ShapeDtypeStruct((B, T, OUTP), jnp.float32),
        scratch_types=[
            pltpu.VMEM((NCHUNK, CHUNK), jnp.int32),
            pltpu.VMEM((2, CHUNK, OUTP), jnp.float32),
            pltpu.SemaphoreType.DMA,
            pltpu.SemaphoreType.DMA,
        ],
    )
    return fn(idx, c)


def kernel(inputs, emb0, emb1, W0, b0, W1, b1, Wa, ba):
    idx = inputs.reshape(NW, NCHUNK, CHUNK).astype(jnp.int32)
    emb0t = emb0.T
    emb1t = emb1.T
    W0p = jnp.pad(W0, ((0, 0), (0, OUTP - OUT)))
    W1p = jnp.pad(W1, ((0, 0), (0, OUTP - OUT)))
    b0p = jnp.pad(b0, (0, OUTP - OUT)).reshape(1, OUTP)
    b1p = jnp.pad(b1, (0, OUTP - OUT)).reshape(1, OUTP)
    wap = jnp.pad(Wa.reshape(1, OUT), ((0, 0), (0, OUTP - OUT)))
    c = _tc_combined_table(emb0t, emb1t, W0p, W1p, b0p, b1p, wap)
    g = _sc_gather(idx, c)
    return g[:, :, :OUT]
